# baseline (speedup 1.0000x reference)
import functools

import jax
import jax.numpy as jnp
from jax import lax
from jax.experimental import pallas as pl
from jax.experimental.pallas import tpu as pltpu
from jax.experimental.pallas import tpu_sc as plsc

B = 4096
D = 64
ITEM_VOCAB = 100000
GENRE_VOCAB = 32
N_GENRES = 8


_NC, _NS = 2, 16
_NW = _NC * _NS
_BPW = B // _NW

@functools.cache
def _sc_gather_fn():
    mesh = plsc.VectorSubcoreMesh(core_axis_name="c", subcore_axis_name="s")

    @functools.partial(
        pl.kernel,
        out_type=jax.ShapeDtypeStruct((B, 2 * D), jnp.float32),
        mesh=mesh,
        scratch_types=[
            pltpu.VMEM((_BPW,), jnp.int32),
            pltpu.VMEM((_BPW,), jnp.int32),
            pltpu.VMEM((_BPW, 2 * D), jnp.float32),
            pltpu.SemaphoreType.DMA,
        ],
    )
    def _sc_gather(idx_hbm, table_hbm, out_hbm, idx_v, pair_v, rows_v, sem):
        wid = lax.axis_index("s") * _NC + lax.axis_index("c")
        base = wid * _BPW
        pltpu.sync_copy(idx_hbm.at[pl.ds(base, _BPW)], idx_v)
        for j in range(_BPW // 16):
            sl = pl.ds(j * 16, 16)
            pair_v[sl] = lax.shift_right_logical(idx_v[sl], 1)
        pltpu.async_copy(table_hbm.at[pair_v], rows_v, sem).wait()
        pltpu.sync_copy(rows_v, out_hbm.at[pl.ds(base, _BPW)])

    return _sc_gather


_BLK = 512


def _tower_body(pair_ref, par_ref, genres_ref, cont_ref, gtab_ref, wc_ref,
                bc_ref, w1_ref, b1_ref, gam_ref, bet_ref, w2_ref, b2_ref,
                out_ref):
    f32 = jnp.float32
    pair = pair_ref[...]
    odd = (par_ref[...] & 1) == 1
    i_emb = jnp.where(odd, pair[:, D:], pair[:, :D])
    genres = genres_ref[...]
    vocab_ids = lax.broadcasted_iota(jnp.int32, (1, GENRE_VOCAB), 1)
    counts = jnp.zeros((_BLK, GENRE_VOCAB), f32)
    for g in range(N_GENRES):
        col = genres[:, g:g + 1]
        counts = counts + ((col == vocab_ids) & (col > 0)).astype(f32)
    g_sum = jnp.dot(counts, gtab_ref[...], preferred_element_type=f32)
    denom = jnp.sum(counts, axis=1, keepdims=True) + 1e-8
    g_emb = g_sum / denom

    cont_emb = jnp.maximum(
        jnp.dot(cont_ref[...], wc_ref[...], preferred_element_type=f32)
        + bc_ref[...], 0.0)

    concat = jnp.concatenate([i_emb, g_emb, cont_emb], axis=1)
    h = jnp.dot(concat, w1_ref[...], preferred_element_type=f32) + b1_ref[...]
    mu = jnp.mean(h, axis=-1, keepdims=True)
    var = jnp.mean((h - mu) ** 2, axis=-1, keepdims=True)
    h = (h - mu) / jnp.sqrt(var + 1e-5) * gam_ref[...] + bet_ref[...]
    h = jnp.maximum(h, 0.0)
    out = jnp.dot(h, w2_ref[...], preferred_element_type=f32) + b2_ref[...]
    norm = jnp.sqrt(jnp.sum(out * out, axis=1, keepdims=True))
    out_ref[...] = out / jnp.maximum(norm, 1e-12)


def _tower(pair, par, genres, cont, gtab, wc, bc, w1, b1, gam, bet, w2, b2):
    fixed = lambda *_: (0, 0)
    row = lambda i: (i, 0)
    return pl.pallas_call(
        _tower_body,
        grid=(B // _BLK,),
        in_specs=[
            pl.BlockSpec((_BLK, 2 * D), row),
            pl.BlockSpec((_BLK, 1), row),
            pl.BlockSpec((_BLK, N_GENRES), row),
            pl.BlockSpec((_BLK, 3), row),
            pl.BlockSpec((GENRE_VOCAB, D), fixed),
            pl.BlockSpec((3, D), fixed),
            pl.BlockSpec((1, D), fixed),
            pl.BlockSpec((3 * D, 2 * D), fixed),
            pl.BlockSpec((1, 2 * D), fixed),
            pl.BlockSpec((1, 2 * D), fixed),
            pl.BlockSpec((1, 2 * D), fixed),
            pl.BlockSpec((2 * D, D), fixed),
            pl.BlockSpec((1, D), fixed),
        ],
        out_specs=pl.BlockSpec((_BLK, D), row),
        out_shape=jax.ShapeDtypeStruct((B, D), jnp.float32),
        compiler_params=pltpu.CompilerParams(
            dimension_semantics=("arbitrary",)),
    )(pair, par, genres, cont, gtab, wc, bc, w1, b1, gam, bet, w2, b2)


def kernel(item_id, tmdb_genres, release_year, avg_rating, revenue,
           item_table, genre_table, W_cont, b_cont, W1, b1,
           ln_gamma, ln_beta, W2, b2):
    item_id = item_id.astype(jnp.int32)
    table2 = item_table.reshape(ITEM_VOCAB // 2, 2 * D)
    pair = _sc_gather_fn()(item_id, table2)
    cont = jnp.stack([release_year, avg_rating, revenue], axis=1)
    return _tower(pair, item_id.reshape(B, 1), tmdb_genres.astype(jnp.int32),
                  cont, genre_table, W_cont, b_cont.reshape(1, D), W1,
                  b1.reshape(1, 2 * D), ln_gamma.reshape(1, 2 * D),
                  ln_beta.reshape(1, 2 * D), W2, b2.reshape(1, D))


# speedup vs baseline: 1.0022x; 1.0022x over previous
import functools

import jax
import jax.numpy as jnp
from jax import lax
from jax.experimental import pallas as pl
from jax.experimental.pallas import tpu as pltpu
from jax.experimental.pallas import tpu_sc as plsc

B = 4096
D = 64
ITEM_VOCAB = 100000
GENRE_VOCAB = 32
N_GENRES = 8


_NC, _NS = 2, 16
_NW = _NC * _NS
_BPW = B // _NW

@functools.cache
def _sc_gather_fn():
    mesh = plsc.VectorSubcoreMesh(core_axis_name="c", subcore_axis_name="s")

    @functools.partial(
        pl.kernel,
        out_type=jax.ShapeDtypeStruct((B, 2 * D), jnp.float32),
        mesh=mesh,
        scratch_types=[
            pltpu.VMEM((_BPW,), jnp.int32),
            pltpu.VMEM((_BPW,), jnp.int32),
            pltpu.VMEM((_BPW, 2 * D), jnp.float32),
            pltpu.SemaphoreType.DMA,
        ],
        compiler_params=pltpu.CompilerParams(use_tc_tiling_on_sc=True),
    )
    def _sc_gather(idx_hbm, table_hbm, out_hbm, idx_v, pair_v, rows_v, sem):
        wid = lax.axis_index("s") * _NC + lax.axis_index("c")
        base = wid * _BPW
        pltpu.sync_copy(idx_hbm.at[pl.ds(base, _BPW)], idx_v)
        for j in range(_BPW // 16):
            sl = pl.ds(j * 16, 16)
            pair_v[sl] = lax.shift_right_logical(idx_v[sl], 1)
        pltpu.async_copy(table_hbm.at[pair_v], rows_v, sem).wait()
        pltpu.sync_copy(rows_v, out_hbm.at[pl.ds(base, _BPW)])

    return _sc_gather


_BLK = 512


def _tower_body(pair_ref, par_ref, genres_ref, cont_ref, gtab_ref, wc_ref,
                bc_ref, w1_ref, b1_ref, gam_ref, bet_ref, w2_ref, b2_ref,
                out_ref):
    f32 = jnp.float32
    pair = pair_ref[...]
    odd = (par_ref[...] & 1) == 1
    i_emb = jnp.where(odd, pair[:, D:], pair[:, :D])
    genres = genres_ref[...]
    vocab_ids = lax.broadcasted_iota(jnp.int32, (1, GENRE_VOCAB), 1)
    counts = jnp.zeros((_BLK, GENRE_VOCAB), f32)
    for g in range(N_GENRES):
        col = genres[:, g:g + 1]
        counts = counts + ((col == vocab_ids) & (col > 0)).astype(f32)
    g_sum = jnp.dot(counts, gtab_ref[...], preferred_element_type=f32)
    denom = jnp.sum(counts, axis=1, keepdims=True) + 1e-8
    g_emb = g_sum / denom

    cont_emb = jnp.maximum(
        jnp.dot(cont_ref[...], wc_ref[...], preferred_element_type=f32)
        + bc_ref[...], 0.0)

    concat = jnp.concatenate([i_emb, g_emb, cont_emb], axis=1)
    h = jnp.dot(concat, w1_ref[...], preferred_element_type=f32) + b1_ref[...]
    mu = jnp.mean(h, axis=-1, keepdims=True)
    var = jnp.mean((h - mu) ** 2, axis=-1, keepdims=True)
    h = (h - mu) / jnp.sqrt(var + 1e-5) * gam_ref[...] + bet_ref[...]
    h = jnp.maximum(h, 0.0)
    out = jnp.dot(h, w2_ref[...], preferred_element_type=f32) + b2_ref[...]
    norm = jnp.sqrt(jnp.sum(out * out, axis=1, keepdims=True))
    out_ref[...] = out / jnp.maximum(norm, 1e-12)


def _tower(pair, par, genres, cont, gtab, wc, bc, w1, b1, gam, bet, w2, b2):
    fixed = lambda *_: (0, 0)
    row = lambda i: (i, 0)
    return pl.pallas_call(
        _tower_body,
        grid=(B // _BLK,),
        in_specs=[
            pl.BlockSpec((_BLK, 2 * D), row),
            pl.BlockSpec((_BLK, 1), row),
            pl.BlockSpec((_BLK, N_GENRES), row),
            pl.BlockSpec((_BLK, 3), row),
            pl.BlockSpec((GENRE_VOCAB, D), fixed),
            pl.BlockSpec((3, D), fixed),
            pl.BlockSpec((1, D), fixed),
            pl.BlockSpec((3 * D, 2 * D), fixed),
            pl.BlockSpec((1, 2 * D), fixed),
            pl.BlockSpec((1, 2 * D), fixed),
            pl.BlockSpec((1, 2 * D), fixed),
            pl.BlockSpec((2 * D, D), fixed),
            pl.BlockSpec((1, D), fixed),
        ],
        out_specs=pl.BlockSpec((_BLK, D), row),
        out_shape=jax.ShapeDtypeStruct((B, D), jnp.float32),
        compiler_params=pltpu.CompilerParams(
            dimension_semantics=("arbitrary",)),
    )(pair, par, genres, cont, gtab, wc, bc, w1, b1, gam, bet, w2, b2)


def kernel(item_id, tmdb_genres, release_year, avg_rating, revenue,
           item_table, genre_table, W_cont, b_cont, W1, b1,
           ln_gamma, ln_beta, W2, b2):
    item_id = item_id.astype(jnp.int32)
    table2 = item_table.reshape(ITEM_VOCAB // 2, 2 * D)
    pair = _sc_gather_fn()(item_id, table2)
    cont = jnp.stack([release_year, avg_rating, revenue], axis=1)
    return _tower(pair, item_id.reshape(B, 1), tmdb_genres.astype(jnp.int32),
                  cont, genre_table, W_cont, b_cont.reshape(1, D), W1,
                  b1.reshape(1, 2 * D), ln_gamma.reshape(1, 2 * D),
                  ln_beta.reshape(1, 2 * D), W2, b2.reshape(1, D))


# speedup vs baseline: 1.4645x; 1.4613x over previous
import functools

import jax
import jax.numpy as jnp
from jax import lax
from jax.experimental import pallas as pl
from jax.experimental.pallas import tpu as pltpu
from jax.experimental.pallas import tpu_sc as plsc

B = 4096
D = 64
ITEM_VOCAB = 100000
GENRE_VOCAB = 32
N_GENRES = 8


_NC, _NS = 2, 16
_NW = _NC * _NS
_DPW = D // _NW


@functools.cache
def _sc_gather_fn():
    mesh = plsc.VectorSubcoreMesh(core_axis_name="c", subcore_axis_name="s")

    @functools.partial(
        pl.kernel,
        out_type=jax.ShapeDtypeStruct((D, B), jnp.float32),
        mesh=mesh,
        scratch_types=[
            pltpu.VMEM((B,), jnp.int32),
            pltpu.VMEM((B,), jnp.float32),
            pltpu.SemaphoreType.DMA,
        ],
        compiler_params=pltpu.CompilerParams(use_tc_tiling_on_sc=False),
    )
    def _sc_gather(idx_hbm, tableT_hbm, outT_hbm, idx_v, row_v, sem):
        wid = lax.axis_index("s") * _NC + lax.axis_index("c")
        pltpu.sync_copy(idx_hbm, idx_v)
        for j in range(_DPW):
            d = wid * _DPW + j
            pltpu.async_copy(tableT_hbm.at[d].at[idx_v], row_v, sem).wait()
            pltpu.sync_copy(row_v, outT_hbm.at[d])

    return _sc_gather


_BLK = 512


def _tower_body(iembT_ref, genT_ref, contT_ref, gtab_ref, wc_ref, bc_ref,
                w1_ref, b1_ref, gam_ref, bet_ref, w2_ref, b2_ref, outT_ref):
    f32 = jnp.float32
    cdim0 = (((0,), (0,)), ((), ()))
    genT = genT_ref[...]
    vocab_ids = lax.broadcasted_iota(jnp.int32, (GENRE_VOCAB, 1), 0)
    countsT = jnp.zeros((GENRE_VOCAB, _BLK), f32)
    for g in range(N_GENRES):
        row = genT[g:g + 1, :]
        countsT = countsT + ((row == vocab_ids) & (row > 0)).astype(f32)
    g_sumT = lax.dot_general(gtab_ref[...], countsT, cdim0,
                             preferred_element_type=f32)
    denom = jnp.sum(countsT, axis=0, keepdims=True) + 1e-8
    g_embT = g_sumT / denom

    cont_embT = jnp.maximum(
        lax.dot_general(wc_ref[...], contT_ref[...], cdim0,
                        preferred_element_type=f32) + bc_ref[...], 0.0)

    concatT = jnp.concatenate([iembT_ref[...], g_embT, cont_embT], axis=0)
    h = lax.dot_general(w1_ref[...], concatT, cdim0,
                        preferred_element_type=f32) + b1_ref[...]
    mu = jnp.mean(h, axis=0, keepdims=True)
    var = jnp.mean((h - mu) ** 2, axis=0, keepdims=True)
    h = (h - mu) / jnp.sqrt(var + 1e-5) * gam_ref[...] + bet_ref[...]
    h = jnp.maximum(h, 0.0)
    outT = lax.dot_general(w2_ref[...], h, cdim0,
                           preferred_element_type=f32) + b2_ref[...]
    norm = jnp.sqrt(jnp.sum(outT * outT, axis=0, keepdims=True))
    outT_ref[...] = outT / jnp.maximum(norm, 1e-12)


def _tower(iembT, genT, contT, gtab, wc, bc, w1, b1, gam, bet, w2, b2):
    fixed = lambda *_: (0, 0)
    col = lambda i: (0, i)
    return pl.pallas_call(
        _tower_body,
        grid=(B // _BLK,),
        in_specs=[
            pl.BlockSpec((D, _BLK), col),
            pl.BlockSpec((N_GENRES, _BLK), col),
            pl.BlockSpec((3, _BLK), col),
            pl.BlockSpec((GENRE_VOCAB, D), fixed),
            pl.BlockSpec((3, D), fixed),
            pl.BlockSpec((D, 1), fixed),
            pl.BlockSpec((3 * D, 2 * D), fixed),
            pl.BlockSpec((2 * D, 1), fixed),
            pl.BlockSpec((2 * D, 1), fixed),
            pl.BlockSpec((2 * D, 1), fixed),
            pl.BlockSpec((2 * D, D), fixed),
            pl.BlockSpec((D, 1), fixed),
        ],
        out_specs=pl.BlockSpec((D, _BLK), col),
        out_shape=jax.ShapeDtypeStruct((D, B), jnp.float32),
        compiler_params=pltpu.CompilerParams(
            dimension_semantics=("arbitrary",)),
    )(iembT, genT, contT, gtab, wc, bc, w1, b1, gam, bet, w2, b2)


def kernel(item_id, tmdb_genres, release_year, avg_rating, revenue,
           item_table, genre_table, W_cont, b_cont, W1, b1,
           ln_gamma, ln_beta, W2, b2):
    item_id = item_id.astype(jnp.int32)
    iembT = _sc_gather_fn()(item_id, item_table.T)
    contT = jnp.stack([release_year, avg_rating, revenue], axis=0)
    outT = _tower(iembT, tmdb_genres.astype(jnp.int32).T, contT,
                  genre_table, W_cont, b_cont.reshape(D, 1), W1,
                  b1.reshape(2 * D, 1), ln_gamma.reshape(2 * D, 1),
                  ln_beta.reshape(2 * D, 1), W2, b2.reshape(D, 1))
    return outT.T
